# baseline (speedup 1.0000x reference)
import functools

import jax
import jax.numpy as jnp
from jax import lax
from jax.experimental import pallas as pl
from jax.experimental.pallas import tpu as pltpu
from jax.experimental.pallas import tpu_sc as plsc

_N_CORES = 2
_N_SUBCORES = 16
_NW = _N_CORES * _N_SUBCORES
_CHUNK = 128
_L = 16


def _plane_gather(b_per_w, n_planes, feat_shape, *refs):
    planes = refs[:n_planes]
    win_hbm, idx_hbm, out = refs[n_planes:n_planes + 3]
    win_i, idx_v, w_v, o_v, sem = refs[n_planes + 3:]

    wid = lax.axis_index("s") * _N_CORES + lax.axis_index("c")
    base = wid * b_per_w
    n_ch = b_per_w // _CHUNK

    copies = []
    for j in range(n_ch):
        pltpu.sync_copy(win_hbm.at[pl.ds(base + j * _CHUNK, _CHUNK)],
                        win_i.at[j])
        for m in range(n_planes):
            copies.append(pltpu.async_copy(
                planes[m].at[win_i.at[j]],
                w_v.at[pl.ds(m * b_per_w + j * _CHUNK, _CHUNK)], sem))
    for j in range(n_ch):
        pltpu.sync_copy(idx_hbm.at[pl.ds(base + j * _CHUNK, _CHUNK)],
                        idx_v.at[j])
    for c in copies:
        c.wait()

    iota = lax.iota(jnp.int32, _L)
    for g in range(b_per_w // _L):
        iv = idx_v[g // (_CHUNK // _L), pl.ds((g % (_CHUNK // _L)) * _L, _L)]
        p_vec = iota + g * _L
        off = iv & 15
        for m in range(n_planes):
            v = plsc.load_gather(w_v, [p_vec + m * b_per_w, off])
            plsc.store_scatter(
                o_v, [jnp.full((_L,), fd, jnp.int32)
                      for fd in feat_shape(m)] + [p_vec], v)

    if len(feat_shape(0)) == 2:
        pltpu.sync_copy(o_v, out.at[:, :, pl.ds(base, b_per_w)])
    else:
        pltpu.sync_copy(o_v, out.at[:, pl.ds(base, b_per_w)])


@jax.jit
def kernel(ind, orientations, translations):
    n_poses = orientations.shape[0]
    batch = ind.shape[0]
    b_per_w = batch // _NW
    nw16 = n_poses // _L

    planes = [orientations[:, a, b].reshape(nw16, _L)
              for a in range(3) for b in range(3)]
    tplanes = [translations[:, a].reshape(nw16, _L) for a in range(2)]
    ind = ind.astype(jnp.int32)
    win = ind >> 4

    mesh = plsc.VectorSubcoreMesh(core_axis_name="c", subcore_axis_name="s")
    cp = pltpu.CompilerParams(use_tc_tiling_on_sc=False,
                              needs_layout_passes=False)
    t_f = pl.kernel(
        functools.partial(_plane_gather, b_per_w, 2, lambda m: (m,)),
        mesh=mesh,
        compiler_params=cp,
        out_type=jax.ShapeDtypeStruct((2, batch), jnp.float32),
        scratch_types=[
            pltpu.VMEM((b_per_w // _CHUNK, _CHUNK), jnp.int32),
            pltpu.VMEM((b_per_w // _CHUNK, _CHUNK), jnp.int32),
            pltpu.VMEM((2 * b_per_w, _L), jnp.float32),
            pltpu.VMEM((2, b_per_w), jnp.float32),
            pltpu.SemaphoreType.DMA,
        ],
    )(*tplanes, win, ind)
    r_parts = []
    for a in range(3):
        grp = lax.optimization_barrier(tuple(planes[3 * a:3 * a + 3]))
        r_parts.append(pl.kernel(
            functools.partial(_plane_gather, b_per_w, 3, lambda m: (m,)),
            mesh=mesh,
            compiler_params=cp,
            out_type=jax.ShapeDtypeStruct((3, batch), jnp.float32),
            scratch_types=[
                pltpu.VMEM((b_per_w // _CHUNK, _CHUNK), jnp.int32),
                pltpu.VMEM((b_per_w // _CHUNK, _CHUNK), jnp.int32),
                pltpu.VMEM((3 * b_per_w, _L), jnp.float32),
                pltpu.VMEM((3, b_per_w), jnp.float32),
                pltpu.SemaphoreType.DMA,
            ],
        )(*grp, win, ind))
    r_f = jnp.stack(r_parts, axis=0)
    return (jnp.transpose(r_f, (2, 0, 1)), jnp.transpose(t_f, (1, 0)))


# speedup vs baseline: 2.7270x; 2.7270x over previous
import functools

import jax
import jax.numpy as jnp
from jax import lax
from jax.experimental import pallas as pl
from jax.experimental.pallas import tpu as pltpu
from jax.experimental.pallas import tpu_sc as plsc

_N_CORES = 2
_N_SUBCORES = 16
_NW = _N_CORES * _N_SUBCORES
_CHUNK = 128
_L = 16


def _plane_gather(b_per_w, n_planes, feat_shape, *refs):
    planes = refs[:n_planes]
    win_hbm, idx_hbm, out = refs[n_planes:n_planes + 3]
    win_i, idx_v, w_v, o_v, sem = refs[n_planes + 3:]

    wid = lax.axis_index("s") * _N_CORES + lax.axis_index("c")
    base = wid * b_per_w
    n_ch = b_per_w // _CHUNK

    copies = []
    for j in range(n_ch):
        pltpu.sync_copy(win_hbm.at[pl.ds(base + j * _CHUNK, _CHUNK)],
                        win_i.at[j])
        for m in range(n_planes):
            copies.append(pltpu.async_copy(
                planes[m].at[win_i.at[j]],
                w_v.at[pl.ds(m * b_per_w + j * _CHUNK, _CHUNK)], sem))
    for j in range(n_ch):
        pltpu.sync_copy(idx_hbm.at[pl.ds(base + j * _CHUNK, _CHUNK)],
                        idx_v.at[j])
    for c in copies:
        c.wait()

    iota = lax.iota(jnp.int32, _L)
    for g in range(b_per_w // _L):
        iv = idx_v[g // (_CHUNK // _L), pl.ds((g % (_CHUNK // _L)) * _L, _L)]
        p_vec = iota + g * _L
        off = iv & 15
        for m in range(n_planes):
            v = plsc.load_gather(w_v, [p_vec + m * b_per_w, off])
            plsc.store_scatter(
                o_v, [jnp.full((_L,), fd, jnp.int32)
                      for fd in feat_shape(m)] + [p_vec], v)

    if len(feat_shape(0)) == 2:
        pltpu.sync_copy(o_v, out.at[:, :, pl.ds(base, b_per_w)])
    else:
        pltpu.sync_copy(o_v, out.at[:, pl.ds(base, b_per_w)])


@jax.jit
def kernel(ind, orientations, translations):
    n_poses = orientations.shape[0]
    batch = ind.shape[0]
    b_per_w = batch // _NW
    nw16 = n_poses // _L

    planes = [orientations[:, a, b].reshape(nw16, _L)
              for a in range(3) for b in range(3)]
    tplanes = [translations[:, a].reshape(nw16, _L) for a in range(2)]
    ind = ind.astype(jnp.int32)
    win = ind >> 4

    mesh = plsc.VectorSubcoreMesh(core_axis_name="c", subcore_axis_name="s")
    cp = pltpu.CompilerParams(use_tc_tiling_on_sc=False,
                              needs_layout_passes=False)
    t_f = pl.kernel(
        functools.partial(_plane_gather, b_per_w, 2, lambda m: (m,)),
        mesh=mesh,
        compiler_params=cp,
        out_type=jax.ShapeDtypeStruct((2, batch), jnp.float32),
        scratch_types=[
            pltpu.VMEM((b_per_w // _CHUNK, _CHUNK), jnp.int32),
            pltpu.VMEM((b_per_w // _CHUNK, _CHUNK), jnp.int32),
            pltpu.VMEM((2 * b_per_w, _L), jnp.float32),
            pltpu.VMEM((2, b_per_w), jnp.float32),
            pltpu.SemaphoreType.DMA,
        ],
    )(*tplanes, win, ind)
    r_f = pl.kernel(
        functools.partial(_plane_gather, b_per_w, 9,
                          lambda m: (m // 3, m % 3)),
        mesh=mesh,
        compiler_params=cp,
        out_type=jax.ShapeDtypeStruct((3, 3, batch), jnp.float32),
        scratch_types=[
            pltpu.VMEM((b_per_w // _CHUNK, _CHUNK), jnp.int32),
            pltpu.VMEM((b_per_w // _CHUNK, _CHUNK), jnp.int32),
            pltpu.VMEM((9 * b_per_w, _L), jnp.float32),
            pltpu.VMEM((3, 3, b_per_w), jnp.float32),
            pltpu.SemaphoreType.DMA,
        ],
    )(*planes, win, ind)
    return (jnp.transpose(r_f, (2, 0, 1)), jnp.transpose(t_f, (1, 0)))


# speedup vs baseline: 2.7500x; 1.0084x over previous
import functools

import jax
import jax.numpy as jnp
from jax import lax
from jax.experimental import pallas as pl
from jax.experimental.pallas import tpu as pltpu
from jax.experimental.pallas import tpu_sc as plsc

_N_CORES = 2
_N_SUBCORES = 16
_NW = _N_CORES * _N_SUBCORES
_CHUNK = 128
_L = 16


def _plane_gather(b_per_w, n_planes, feat_shape, *refs):
    planes = refs[:n_planes]
    idx_hbm, out = refs[n_planes:n_planes + 2]
    win_i, idx_v, w_v, o_v, sem = refs[n_planes + 2:]

    wid = lax.axis_index("s") * _N_CORES + lax.axis_index("c")
    base = wid * b_per_w
    n_ch = b_per_w // _CHUNK

    copies = []
    for j in range(n_ch):
        pltpu.sync_copy(idx_hbm.at[pl.ds(base + j * _CHUNK, _CHUNK)],
                        idx_v.at[j])
        for k in range(_CHUNK // _L):
            sl = pl.ds(k * _L, _L)
            win_i[j, sl] = lax.shift_right_logical(idx_v[j, sl], 4)
        for m in range(n_planes):
            copies.append(pltpu.async_copy(
                planes[m].at[win_i.at[j]],
                w_v.at[pl.ds(m * b_per_w + j * _CHUNK, _CHUNK)], sem))
    for c in copies:
        c.wait()

    iota = lax.iota(jnp.int32, _L)
    for g in range(b_per_w // _L):
        iv = idx_v[g // (_CHUNK // _L), pl.ds((g % (_CHUNK // _L)) * _L, _L)]
        p_vec = iota + g * _L
        off = iv & 15
        for m in range(n_planes):
            v = plsc.load_gather(w_v, [p_vec + m * b_per_w, off])
            plsc.store_scatter(
                o_v, [jnp.full((_L,), fd, jnp.int32)
                      for fd in feat_shape(m)] + [p_vec], v)

    if len(feat_shape(0)) == 2:
        pltpu.sync_copy(o_v, out.at[:, :, pl.ds(base, b_per_w)])
    else:
        pltpu.sync_copy(o_v, out.at[:, pl.ds(base, b_per_w)])


@jax.jit
def kernel(ind, orientations, translations):
    n_poses = orientations.shape[0]
    batch = ind.shape[0]
    b_per_w = batch // _NW
    nw16 = n_poses // _L

    planes = [orientations[:, a, b].reshape(nw16, _L)
              for a in range(3) for b in range(3)]
    tplanes = [translations[:, a].reshape(nw16, _L) for a in range(2)]
    ind = ind.astype(jnp.int32)

    mesh = plsc.VectorSubcoreMesh(core_axis_name="c", subcore_axis_name="s")
    cp = pltpu.CompilerParams(use_tc_tiling_on_sc=False,
                              needs_layout_passes=False)
    t_f = pl.kernel(
        functools.partial(_plane_gather, b_per_w, 2, lambda m: (m,)),
        mesh=mesh,
        compiler_params=cp,
        out_type=jax.ShapeDtypeStruct((2, batch), jnp.float32),
        scratch_types=[
            pltpu.VMEM((b_per_w // _CHUNK, _CHUNK), jnp.int32),
            pltpu.VMEM((b_per_w // _CHUNK, _CHUNK), jnp.int32),
            pltpu.VMEM((2 * b_per_w, _L), jnp.float32),
            pltpu.VMEM((2, b_per_w), jnp.float32),
            pltpu.SemaphoreType.DMA,
        ],
    )(*tplanes, ind)
    r_f = pl.kernel(
        functools.partial(_plane_gather, b_per_w, 9,
                          lambda m: (m // 3, m % 3)),
        mesh=mesh,
        compiler_params=cp,
        out_type=jax.ShapeDtypeStruct((3, 3, batch), jnp.float32),
        scratch_types=[
            pltpu.VMEM((b_per_w // _CHUNK, _CHUNK), jnp.int32),
            pltpu.VMEM((b_per_w // _CHUNK, _CHUNK), jnp.int32),
            pltpu.VMEM((9 * b_per_w, _L), jnp.float32),
            pltpu.VMEM((3, 3, b_per_w), jnp.float32),
            pltpu.SemaphoreType.DMA,
        ],
    )(*planes, ind)
    return (jnp.transpose(r_f, (2, 0, 1)), jnp.transpose(t_f, (1, 0)))
